# baseline (speedup 1.0000x reference)
import jax
import jax.numpy as jnp
from jax.experimental import pallas as pl
from jax.experimental.pallas import tpu as pltpu

_N = 20000
_NUM_CLASSES = 81
_NMS_IOU = 0.5
_MAX_OUT = 200
_SCORE_THR = 0.05
_NEG = -1e30
_LANES = 128
_ROWS = (_N + _LANES - 1) // _LANES
_NPAD = _ROWS * _LANES


def _nms_kernel(boxes_ref, confs_ref, out_ref,
                x1_ref, y1_ref, x2_ref, y2_ref, cat_ref, a2_ref,
                work_ref, rm_ref):
    confs = confs_ref[...]
    scores = jnp.max(confs, axis=1)
    cls_iota = jax.lax.broadcasted_iota(jnp.int32, confs.shape, 1)
    cat = jnp.min(
        jnp.where(confs == scores[:, None, :], cls_iota, _NUM_CLASSES), axis=1
    )
    valid = jnp.logical_and(scores > _SCORE_THR, cat != 0)
    work0 = jnp.where(valid, scores, _NEG)

    x1 = boxes_ref[:, 0, :]
    y1 = boxes_ref[:, 1, :]
    x2 = boxes_ref[:, 2, :]
    y2 = boxes_ref[:, 3, :]
    max_coord = jnp.max(jnp.maximum(jnp.maximum(x1, y1), jnp.maximum(x2, y2)))
    catf = cat.astype(jnp.float32)
    off = catf * (max_coord + 1.0)
    x1_ref[...] = x1 + off
    y1_ref[...] = y1 + off
    x2_ref[...] = x2 + off
    y2_ref[...] = y2 + off
    cat_ref[...] = catf
    a2_ref[...] = (x2_ref[...] - x1_ref[...]) * (y2_ref[...] - y1_ref[...])
    work_ref[...] = work0
    rm_ref[...] = jnp.max(work0, axis=1, keepdims=True)

    lane = jax.lax.broadcasted_iota(jnp.int32, (1, _LANES), 1)
    rowi = jax.lax.broadcasted_iota(jnp.int32, (_ROWS, 1), 0)
    link = (
        jax.lax.broadcasted_iota(jnp.int32, (8, _LANES), 0) * _LANES
        + jax.lax.broadcasted_iota(jnp.int32, (8, _LANES), 1)
    )

    def _pick(ref, r, onehot):
        return jnp.sum(jnp.where(onehot, ref[pl.ds(r, 1), :], 0.0))

    kzero = jnp.zeros((8, _LANES), jnp.float32)

    def cond(carry):
        k, m, _, _, _, _, _ = carry
        return jnp.logical_and(k < _MAX_OUT, m > _NEG / 2)

    def body(carry):
        k, m, kx1, ky1, kx2, ky2, ka = carry
        r = jnp.min(jnp.where(rm_ref[...] == m, rowi, _ROWS))
        row = work_ref[pl.ds(r, 1), :]
        lf = jnp.min(jnp.where(row == m, lane, _LANES))
        onehot = lane == lf
        x1s = _pick(x1_ref, r, onehot)
        y1s = _pick(y1_ref, r, onehot)
        x2s = _pick(x2_ref, r, onehot)
        y2s = _pick(y2_ref, r, onehot)
        cs = _pick(cat_ref, r, onehot)
        a2s = _pick(a2_ref, r, onehot)
        ltx = jnp.maximum(kx1, x1s)
        lty = jnp.maximum(ky1, y1s)
        rbx = jnp.minimum(kx2, x2s)
        rby = jnp.minimum(ky2, y2s)
        w = jnp.maximum(rbx - ltx, 0.0)
        h = jnp.maximum(rby - lty, 0.0)
        inter = w * h
        iou = inter / (ka + a2s - inter + 1e-9)
        keep = jnp.max(iou) <= _NMS_IOU
        kf = keep.astype(jnp.float32)
        sel = link == k
        kx1 = jnp.where(sel, x1s * kf, kx1)
        ky1 = jnp.where(sel, y1s * kf, ky1)
        kx2 = jnp.where(sel, x2s * kf, kx2)
        ky2 = jnp.where(sel, y2s * kf, ky2)
        ka = jnp.where(sel, ((x2s - x1s) * (y2s - y1s)) * kf, ka)
        offs = cs * (max_coord + 1.0)
        vals = (x1s - offs, y1s - offs, x2s - offs, y2s - offs, m, cs)
        orow = jnp.zeros((1, _LANES), jnp.float32)
        for j, v in enumerate(vals):
            orow = orow + jnp.where(lane == j, v, 0.0)
        out_ref[pl.ds(k, 1), :] = orow
        row2 = jnp.where(onehot, _NEG, row)
        work_ref[pl.ds(r, 1), :] = row2
        rm_ref[pl.ds(r, 1), :] = jnp.max(row2).reshape(1, 1)
        m2 = jnp.max(rm_ref[...])
        return (k + keep.astype(jnp.int32), m2, kx1, ky1, kx2, ky2, ka)

    m0 = jnp.max(rm_ref[...])
    init = (jnp.int32(0), m0, kzero, kzero, kzero, kzero, kzero)
    kfin = jax.lax.while_loop(cond, body, init)[0]
    oi = jax.lax.broadcasted_iota(jnp.int32, (_MAX_OUT, _LANES), 0)
    out_ref[...] = jnp.where(oi < kfin, out_ref[...], 0.0)


def _run(boxes3, confs3, interpret=False):
    return pl.pallas_call(
        _nms_kernel,
        out_shape=jax.ShapeDtypeStruct((_MAX_OUT, _LANES), jnp.float32),
        scratch_shapes=[pltpu.VMEM((_ROWS, _LANES), jnp.float32)] * 7
        + [pltpu.VMEM((_ROWS, 1), jnp.float32)],
        interpret=interpret,
    )(boxes3, confs3)


def kernel(boxes, confs, max_output):
    boxes_p = jnp.pad(boxes, ((0, _NPAD - _N), (0, 0)))
    confs_p = jnp.pad(confs, ((0, _NPAD - _N), (0, 0)), constant_values=-1.0)
    boxes3 = boxes_p.reshape(_ROWS, _LANES, 4).transpose(0, 2, 1)
    confs3 = confs_p.reshape(_ROWS, _LANES, _NUM_CLASSES).transpose(0, 2, 1)
    out = _run(boxes3, confs3)
    mask = jnp.arange(_MAX_OUT) < max_output
    mf = mask.astype(jnp.float32)
    boxes_out = out[:, 0:4] * mf[:, None]
    cats_out = jnp.where(mask, out[:, 5].astype(jnp.int32), 0)
    scores_out = out[:, 4] * mf
    return boxes_out, cats_out, scores_out
